# Initial kernel scaffold; baseline (speedup 1.0000x reference)
import jax
import jax.numpy as jnp
from jax.experimental import pallas as pl


def kernel(x, edge_index, W_neigh1, W_self1, b1, W_neigh2, W_self2, b2, W_neigh3, W_self3, b3):
    raise NotImplementedError("write your pallas kernel here")


# speedup vs baseline: 4.7540x; 4.7540x over previous
import functools

import jax
import jax.numpy as jnp
from jax import lax
from jax.experimental import pallas as pl
from jax.experimental.pallas import tpu as pltpu
import jax.experimental.pallas.tpu_sc as plsc

N_NODES = 10000
N_EDGES = 320000
D = 128

NC = 2
NS = 16
NW = NC * NS
EPW = N_EDGES // NW
K = 80
N_CHUNKS = EPW // K
CP = 624
TAIL = N_NODES - NS * CP


def _zero_acc(z2_hbm, acc_sh, s):
    pltpu.sync_copy(z2_hbm.at[pl.ds(0, CP)], acc_sh.at[pl.ds(s * CP, CP)])

    @pl.when(s == NS - 1)
    def _():
        pltpu.sync_copy(z2_hbm.at[pl.ds(CP, TAIL)],
                        acc_sh.at[pl.ds(NS * CP, TAIL)])


def _drain_acc(acc_sh, out_hbm, c, s):
    pltpu.sync_copy(acc_sh.at[pl.ds(s * CP, CP)],
                    out_hbm.at[c, pl.ds(s * CP, CP)])

    @pl.when(s == NS - 1)
    def _():
        pltpu.sync_copy(acc_sh.at[pl.ds(NS * CP, TAIL)],
                        out_hbm.at[c, pl.ds(NS * CP, TAIL)])


def _sc_agg_body(h_hbm, src_hbm, dst_hbm, z2_hbm, out_hbm,
                 acc_sh, sidx, didx, rows, sem):
    c = lax.axis_index("c")
    s = lax.axis_index("s")
    wid = s * NC + c
    base = wid * EPW

    _zero_acc(z2_hbm, acc_sh, s)
    plsc.subcore_barrier()

    def body(i, carry):
        off = base + i * K
        pltpu.sync_copy(src_hbm.at[pl.ds(off, K)], sidx)
        pltpu.sync_copy(dst_hbm.at[pl.ds(off, K)], didx)
        pltpu.async_copy(h_hbm.at[sidx], rows, sem).wait()
        pltpu.sync_copy(rows, acc_sh.at[didx], add=True)
        return carry

    lax.fori_loop(0, N_CHUNKS, body, 0)
    plsc.subcore_barrier()
    _drain_acc(acc_sh, out_hbm, c, s)


def _sc_deg_body(dst_hbm, ones_hbm, z2_hbm, out_hbm, acc_sh, didx, rows):
    c = lax.axis_index("c")
    s = lax.axis_index("s")
    wid = s * NC + c
    base = wid * EPW

    _zero_acc(z2_hbm, acc_sh, s)
    pltpu.sync_copy(ones_hbm, rows)
    plsc.subcore_barrier()

    def body(i, carry):
        off = base + i * K
        pltpu.sync_copy(dst_hbm.at[pl.ds(off, K)], didx)
        pltpu.sync_copy(rows, acc_sh.at[didx], add=True)
        return carry

    lax.fori_loop(0, N_CHUNKS, body, 0)
    plsc.subcore_barrier()
    _drain_acc(acc_sh, out_hbm, c, s)


_MESH = dict(core_axis_name="c", subcore_axis_name="s")
_ACC_OUT = [jax.ShapeDtypeStruct((NC, N_NODES, D), jnp.float32)]


def _make_sc_agg():
    return pl.kernel(
        _sc_agg_body,
        out_type=_ACC_OUT,
        mesh=plsc.VectorSubcoreMesh(**_MESH),
        scratch_types=[
            pltpu.VMEM_SHARED((N_NODES, D), jnp.float32),
            pltpu.VMEM((K,), jnp.int32),
            pltpu.VMEM((K,), jnp.int32),
            pltpu.VMEM((K, D), jnp.float32),
            pltpu.SemaphoreType.DMA,
        ])


def _make_sc_deg():
    return pl.kernel(
        _sc_deg_body,
        out_type=_ACC_OUT,
        mesh=plsc.VectorSubcoreMesh(**_MESH),
        scratch_types=[
            pltpu.VMEM_SHARED((N_NODES, D), jnp.float32),
            pltpu.VMEM((K,), jnp.int32),
            pltpu.VMEM((K, D), jnp.float32),
        ])


def _dense_body(relu, h_ref, acc_ref, deg_ref, ws_ref, wn_ref, b_ref, o_ref):
    h = h_ref[...]
    a = acc_ref[0] + acc_ref[1]
    dsum = deg_ref[0, :, 0] + deg_ref[1, :, 0]
    r = (1.0 / jnp.maximum(dsum, 1.0))[:, None]
    hn = a * r
    out = (jnp.dot(h, ws_ref[...], preferred_element_type=jnp.float32)
           + jnp.dot(hn, wn_ref[...], preferred_element_type=jnp.float32)
           + b_ref[...][None, :])
    o_ref[...] = jnp.maximum(out, 0.0) if relu else out


def _dense(h, acc, deg, W_self, W_neigh, b, relu):
    R = 1000
    grid = (N_NODES // R,)
    return pl.pallas_call(
        functools.partial(_dense_body, relu),
        grid=grid,
        in_specs=[
            pl.BlockSpec((R, D), lambda i: (i, 0)),
            pl.BlockSpec((NC, R, D), lambda i: (0, i, 0)),
            pl.BlockSpec((NC, R, D), lambda i: (0, i, 0)),
            pl.BlockSpec((D, D), lambda i: (0, 0)),
            pl.BlockSpec((D, D), lambda i: (0, 0)),
            pl.BlockSpec((D,), lambda i: (0,)),
        ],
        out_specs=pl.BlockSpec((R, D), lambda i: (i, 0)),
        out_shape=jax.ShapeDtypeStruct((N_NODES, D), jnp.float32),
    )(h, acc, deg, W_self, W_neigh, b)


def kernel(x, edge_index, W_neigh1, W_self1, b1, W_neigh2, W_self2, b2,
           W_neigh3, W_self3, b3):
    src = edge_index[0]
    dst = edge_index[1]
    z2 = jnp.zeros((CP + TAIL, D), jnp.float32)
    ones_rows = jnp.ones((K, D), jnp.float32)

    agg = _make_sc_agg()
    degk = _make_sc_deg()

    degacc, = degk(dst, ones_rows, z2)
    acc1, = agg(x, src, dst, z2)
    h1 = _dense(x, acc1, degacc, W_self1, W_neigh1, b1, relu=True)
    acc2, = agg(h1, src, dst, z2)
    h2 = _dense(h1, acc2, degacc, W_self2, W_neigh2, b2, relu=True)
    acc3, = agg(h2, src, dst, z2)
    return _dense(h2, acc3, degacc, W_self3, W_neigh3, b3, relu=False)


# speedup vs baseline: 8.9413x; 1.8808x over previous
import functools

import jax
import jax.numpy as jnp
from jax import lax
from jax.experimental import pallas as pl
from jax.experimental.pallas import tpu as pltpu
import jax.experimental.pallas.tpu_sc as plsc

N_NODES = 10000
N_EDGES = 320000
D = 128

NC = 2
NS = 16
NW = NC * NS
EPW = N_EDGES // NW
K = 125
G = 4
CPG = 20
CP = 624
TAIL = N_NODES - NS * CP


def _zero_acc(z2_hbm, acc_sh, s):
    pltpu.sync_copy(z2_hbm.at[pl.ds(0, CP)], acc_sh.at[pl.ds(s * CP, CP)])

    @pl.when(s == NS - 1)
    def _():
        pltpu.sync_copy(z2_hbm.at[pl.ds(CP, TAIL)],
                        acc_sh.at[pl.ds(NS * CP, TAIL)])


def _drain_acc(acc_sh, out_hbm, c, s):
    pltpu.sync_copy(acc_sh.at[pl.ds(s * CP, CP)],
                    out_hbm.at[c, pl.ds(s * CP, CP)])

    @pl.when(s == NS - 1)
    def _():
        pltpu.sync_copy(acc_sh.at[pl.ds(NS * CP, TAIL)],
                        out_hbm.at[c, pl.ds(NS * CP, TAIL)])


def _sc_agg_body(h_hbm, src_hbm, dst_hbm, z2_hbm, out_hbm, acc_sh,
                 sidx_all, didx_all, rows0, rows1, gs0, gs1, ss0, ss1):
    c = lax.axis_index("c")
    s = lax.axis_index("s")
    wid = s * NC + c

    _zero_acc(z2_hbm, acc_sh, s)
    plsc.subcore_barrier()

    def gather(i, rows, sem):
        return pltpu.async_copy(h_hbm.at[sidx_all.at[i]], rows, sem)

    def scatter(i, rows, sem):
        return pltpu.async_copy(rows, acc_sh.at[didx_all.at[i]], sem, add=True)

    def wait_gather(i, rows, sem):
        pltpu.make_async_copy(h_hbm.at[sidx_all.at[i]], rows, sem).wait()

    def wait_scatter(i, rows, sem):
        pltpu.make_async_copy(rows, acc_sh.at[didx_all.at[i]], sem).wait()

    def group(g, carry):
        pltpu.sync_copy(src_hbm.at[wid, g], sidx_all)
        pltpu.sync_copy(dst_hbm.at[wid, g], didx_all)
        gather(0, rows0, gs0)
        gather(1, rows1, gs1)

        def body(j, carry2):
            a = 2 * j
            b = a + 1
            wait_gather(a, rows0, gs0)
            scatter(a, rows0, ss0)
            wait_gather(b, rows1, gs1)
            scatter(b, rows1, ss1)
            wait_scatter(a, rows0, ss0)

            @pl.when(a + 2 < CPG)
            def _():
                gather(a + 2, rows0, gs0)
            wait_scatter(b, rows1, ss1)

            @pl.when(b + 2 < CPG)
            def _():
                gather(b + 2, rows1, gs1)
            return carry2

        lax.fori_loop(0, CPG // 2, body, 0)
        return carry

    lax.fori_loop(0, G, group, 0)
    plsc.subcore_barrier()
    _drain_acc(acc_sh, out_hbm, c, s)


def _sc_deg_body(dst_hbm, ones_hbm, z2_hbm, out_hbm, acc_sh, didx_all, rows,
                 ss0, ss1):
    c = lax.axis_index("c")
    s = lax.axis_index("s")
    wid = s * NC + c

    _zero_acc(z2_hbm, acc_sh, s)
    pltpu.sync_copy(ones_hbm, rows)
    plsc.subcore_barrier()

    def scatter(i, sem):
        return pltpu.async_copy(rows, acc_sh.at[didx_all.at[i]], sem, add=True)

    def wait_scatter(i, sem):
        pltpu.make_async_copy(rows, acc_sh.at[didx_all.at[i]], sem).wait()

    def group(g, carry):
        pltpu.sync_copy(dst_hbm.at[wid, g], didx_all)

        def body(j, carry2):
            a = 2 * j
            b = a + 1
            scatter(a, ss0)
            scatter(b, ss1)
            wait_scatter(a, ss0)
            wait_scatter(b, ss1)
            return carry2

        lax.fori_loop(0, CPG // 2, body, 0)
        return carry

    lax.fori_loop(0, G, group, 0)
    plsc.subcore_barrier()
    _drain_acc(acc_sh, out_hbm, c, s)


_MESH = dict(core_axis_name="c", subcore_axis_name="s")
_ACC_OUT = [jax.ShapeDtypeStruct((NC, N_NODES, D), jnp.float32)]


def _make_sc_agg():
    return pl.kernel(
        _sc_agg_body,
        out_type=_ACC_OUT,
        mesh=plsc.VectorSubcoreMesh(**_MESH),
        scratch_types=[
            pltpu.VMEM_SHARED((N_NODES, D), jnp.float32),
            pltpu.VMEM((CPG, K), jnp.int32),
            pltpu.VMEM((CPG, K), jnp.int32),
            pltpu.VMEM((K, D), jnp.float32),
            pltpu.VMEM((K, D), jnp.float32),
            pltpu.SemaphoreType.DMA,
            pltpu.SemaphoreType.DMA,
            pltpu.SemaphoreType.DMA,
            pltpu.SemaphoreType.DMA,
        ])


def _make_sc_deg():
    return pl.kernel(
        _sc_deg_body,
        out_type=_ACC_OUT,
        mesh=plsc.VectorSubcoreMesh(**_MESH),
        scratch_types=[
            pltpu.VMEM_SHARED((N_NODES, D), jnp.float32),
            pltpu.VMEM((CPG, K), jnp.int32),
            pltpu.VMEM((K, D), jnp.float32),
            pltpu.SemaphoreType.DMA,
            pltpu.SemaphoreType.DMA,
        ])


def _dense_body(relu, h_ref, acc_ref, deg_ref, ws_ref, wn_ref, b_ref, o_ref):
    h = h_ref[...]
    a = acc_ref[0] + acc_ref[1]
    dsum = deg_ref[0, :, 0] + deg_ref[1, :, 0]
    r = (1.0 / jnp.maximum(dsum, 1.0))[:, None]
    hn = a * r
    out = (jnp.dot(h, ws_ref[...], preferred_element_type=jnp.float32)
           + jnp.dot(hn, wn_ref[...], preferred_element_type=jnp.float32)
           + b_ref[...][None, :])
    o_ref[...] = jnp.maximum(out, 0.0) if relu else out


def _dense(h, acc, deg, W_self, W_neigh, b, relu):
    R = 1000
    grid = (N_NODES // R,)
    return pl.pallas_call(
        functools.partial(_dense_body, relu),
        grid=grid,
        in_specs=[
            pl.BlockSpec((R, D), lambda i: (i, 0)),
            pl.BlockSpec((NC, R, D), lambda i: (0, i, 0)),
            pl.BlockSpec((NC, R, D), lambda i: (0, i, 0)),
            pl.BlockSpec((D, D), lambda i: (0, 0)),
            pl.BlockSpec((D, D), lambda i: (0, 0)),
            pl.BlockSpec((D,), lambda i: (0,)),
        ],
        out_specs=pl.BlockSpec((R, D), lambda i: (i, 0)),
        out_shape=jax.ShapeDtypeStruct((N_NODES, D), jnp.float32),
    )(h, acc, deg, W_self, W_neigh, b)


def kernel(x, edge_index, W_neigh1, W_self1, b1, W_neigh2, W_self2, b2,
           W_neigh3, W_self3, b3):
    src = edge_index[0].reshape(NW, G, CPG, K)
    dst = edge_index[1].reshape(NW, G, CPG, K)
    z2 = jnp.zeros((CP + TAIL, D), jnp.float32)
    ones_rows = jnp.ones((K, D), jnp.float32)

    agg = _make_sc_agg()
    degk = _make_sc_deg()

    degacc, = degk(dst, ones_rows, z2)
    acc1, = agg(x, src, dst, z2)
    h1 = _dense(x, acc1, degacc, W_self1, W_neigh1, b1, relu=True)
    acc2, = agg(h1, src, dst, z2)
    h2 = _dense(h1, acc2, degacc, W_self2, W_neigh2, b2, relu=True)
    acc3, = agg(h2, src, dst, z2)
    return _dense(h2, acc3, degacc, W_self3, W_neigh3, b3, relu=False)


# speedup vs baseline: 10.3074x; 1.1528x over previous
import functools

import jax
import jax.numpy as jnp
from jax import lax
from jax.experimental import pallas as pl
from jax.experimental.pallas import tpu as pltpu
import jax.experimental.pallas.tpu_sc as plsc

N_NODES = 10000
N_EDGES = 320000
D = 128

NC = 2
NS = 16
NW = NC * NS
EPW = N_EDGES // NW
K = 50
G = 5
CPG = 40
NBUF = 4
CP = 624
TAIL = N_NODES - NS * CP


def _zero_acc(z2_hbm, acc_sh, s):
    pltpu.sync_copy(z2_hbm.at[pl.ds(0, CP)], acc_sh.at[pl.ds(s * CP, CP)])

    @pl.when(s == NS - 1)
    def _():
        pltpu.sync_copy(z2_hbm.at[pl.ds(CP, TAIL)],
                        acc_sh.at[pl.ds(NS * CP, TAIL)])


def _drain_acc(acc_sh, out_hbm, c, s):
    pltpu.sync_copy(acc_sh.at[pl.ds(s * CP, CP)],
                    out_hbm.at[c, pl.ds(s * CP, CP)])

    @pl.when(s == NS - 1)
    def _():
        pltpu.sync_copy(acc_sh.at[pl.ds(NS * CP, TAIL)],
                        out_hbm.at[c, pl.ds(NS * CP, TAIL)])


def _sc_agg_body(h_hbm, src_hbm, dst_hbm, z2_hbm, out_hbm, acc_sh,
                 sidxA, sidxB, didxA, didxB, rows0, rows1, rows2, rows3,
                 isS, isD, gs0, gs1, gs2, gs3, ss0, ss1, ss2, ss3):
    c = lax.axis_index("c")
    s = lax.axis_index("s")
    wid = s * NC + c

    _zero_acc(z2_hbm, acc_sh, s)
    plsc.subcore_barrier()

    sidx = (sidxA, sidxB)
    didx = (didxA, didxB)
    bufs = ((rows0, gs0, ss0), (rows1, gs1, ss1),
            (rows2, gs2, ss2), (rows3, gs3, ss3))

    def gather(si, i, rows, sem):
        return pltpu.async_copy(h_hbm.at[si.at[i]], rows, sem)

    def wait_gather(si, i, rows, sem):
        pltpu.make_async_copy(h_hbm.at[si.at[i]], rows, sem).wait()

    def scatter(di, i, rows, sem):
        return pltpu.async_copy(rows, acc_sh.at[di.at[i]], sem, add=True)

    def wait_scatter(di, i, rows, sem):
        pltpu.make_async_copy(rows, acc_sh.at[di.at[i]], sem).wait()

    pltpu.async_copy(src_hbm.at[wid, 0], sidx[0], isS)
    pltpu.async_copy(dst_hbm.at[wid, 0], didx[0], isD)

    for g in range(G):
        si = sidx[g % 2]
        di = didx[g % 2]
        pltpu.make_async_copy(src_hbm.at[wid, g], si, isS).wait()
        pltpu.make_async_copy(dst_hbm.at[wid, g], di, isD).wait()
        if g + 1 < G:
            pltpu.async_copy(src_hbm.at[wid, g + 1], sidx[(g + 1) % 2], isS)
            pltpu.async_copy(dst_hbm.at[wid, g + 1], didx[(g + 1) % 2], isD)

        for b, (rows, gsem, _) in enumerate(bufs):
            gather(si, b, rows, gsem)

        def body(j, carry, si=si, di=di):
            c0 = NBUF * j
            for b, (rows, gsem, ssem) in enumerate(bufs):
                wait_gather(si, c0 + b, rows, gsem)
                scatter(di, c0 + b, rows, ssem)
            for b, (rows, gsem, ssem) in enumerate(bufs):
                wait_scatter(di, c0 + b, rows, ssem)

                @pl.when(c0 + b + NBUF < CPG)
                def _(b=b, rows=rows, gsem=gsem):
                    gather(si, c0 + b + NBUF, rows, gsem)
            return carry

        lax.fori_loop(0, CPG // NBUF, body, 0)

    plsc.subcore_barrier()
    _drain_acc(acc_sh, out_hbm, c, s)


def _sc_deg_body(dst_hbm, ones_hbm, z2_hbm, out_hbm, acc_sh, didx_all, rows,
                 ss0, ss1):
    c = lax.axis_index("c")
    s = lax.axis_index("s")
    wid = s * NC + c

    _zero_acc(z2_hbm, acc_sh, s)
    pltpu.sync_copy(ones_hbm, rows)
    plsc.subcore_barrier()

    def scatter(i, sem):
        return pltpu.async_copy(rows, acc_sh.at[didx_all.at[i]], sem, add=True)

    def wait_scatter(i, sem):
        pltpu.make_async_copy(rows, acc_sh.at[didx_all.at[i]], sem).wait()

    def group(g, carry):
        pltpu.sync_copy(dst_hbm.at[wid, g], didx_all)

        def body(j, carry2):
            a = 2 * j
            b = a + 1
            scatter(a, ss0)
            scatter(b, ss1)
            wait_scatter(a, ss0)
            wait_scatter(b, ss1)
            return carry2

        lax.fori_loop(0, CPG // 2, body, 0)
        return carry

    lax.fori_loop(0, G, group, 0)
    plsc.subcore_barrier()
    _drain_acc(acc_sh, out_hbm, c, s)


_MESH = dict(core_axis_name="c", subcore_axis_name="s")
_ACC_OUT = [jax.ShapeDtypeStruct((NC, N_NODES, D), jnp.float32)]


def _make_sc_agg():
    return pl.kernel(
        _sc_agg_body,
        out_type=_ACC_OUT,
        mesh=plsc.VectorSubcoreMesh(**_MESH),
        scratch_types=[
            pltpu.VMEM_SHARED((N_NODES, D), jnp.float32),
            pltpu.VMEM((CPG, K), jnp.int32),
            pltpu.VMEM((CPG, K), jnp.int32),
            pltpu.VMEM((CPG, K), jnp.int32),
            pltpu.VMEM((CPG, K), jnp.int32),
            pltpu.VMEM((K, D), jnp.float32),
            pltpu.VMEM((K, D), jnp.float32),
            pltpu.VMEM((K, D), jnp.float32),
            pltpu.VMEM((K, D), jnp.float32),
        ] + [pltpu.SemaphoreType.DMA] * 10)


def _make_sc_deg():
    return pl.kernel(
        _sc_deg_body,
        out_type=_ACC_OUT,
        mesh=plsc.VectorSubcoreMesh(**_MESH),
        scratch_types=[
            pltpu.VMEM_SHARED((N_NODES, D), jnp.float32),
            pltpu.VMEM((CPG, K), jnp.int32),
            pltpu.VMEM((K, D), jnp.float32),
            pltpu.SemaphoreType.DMA,
            pltpu.SemaphoreType.DMA,
        ])


def _dense_body(relu, h_ref, acc_ref, deg_ref, ws_ref, wn_ref, b_ref, o_ref):
    h = h_ref[...]
    a = acc_ref[0] + acc_ref[1]
    dsum = deg_ref[0, :, 0] + deg_ref[1, :, 0]
    r = (1.0 / jnp.maximum(dsum, 1.0))[:, None]
    hn = a * r
    out = (jnp.dot(h, ws_ref[...], preferred_element_type=jnp.float32)
           + jnp.dot(hn, wn_ref[...], preferred_element_type=jnp.float32)
           + b_ref[...][None, :])
    o_ref[...] = jnp.maximum(out, 0.0) if relu else out


def _dense(h, acc, deg, W_self, W_neigh, b, relu):
    R = 1000
    grid = (N_NODES // R,)
    return pl.pallas_call(
        functools.partial(_dense_body, relu),
        grid=grid,
        in_specs=[
            pl.BlockSpec((R, D), lambda i: (i, 0)),
            pl.BlockSpec((NC, R, D), lambda i: (0, i, 0)),
            pl.BlockSpec((NC, R, D), lambda i: (0, i, 0)),
            pl.BlockSpec((D, D), lambda i: (0, 0)),
            pl.BlockSpec((D, D), lambda i: (0, 0)),
            pl.BlockSpec((D,), lambda i: (0,)),
        ],
        out_specs=pl.BlockSpec((R, D), lambda i: (i, 0)),
        out_shape=jax.ShapeDtypeStruct((N_NODES, D), jnp.float32),
    )(h, acc, deg, W_self, W_neigh, b)


def kernel(x, edge_index, W_neigh1, W_self1, b1, W_neigh2, W_self2, b2,
           W_neigh3, W_self3, b3):
    src = edge_index[0].reshape(NW, G, CPG, K)
    dst = edge_index[1].reshape(NW, G, CPG, K)
    z2 = jnp.zeros((CP + TAIL, D), jnp.float32)
    ones_rows = jnp.ones((K, D), jnp.float32)

    agg = _make_sc_agg()
    degk = _make_sc_deg()

    degacc, = degk(dst, ones_rows, z2)
    acc1, = agg(x, src, dst, z2)
    h1 = _dense(x, acc1, degacc, W_self1, W_neigh1, b1, relu=True)
    acc2, = agg(h1, src, dst, z2)
    h2 = _dense(h1, acc2, degacc, W_self2, W_neigh2, b2, relu=True)
    acc3, = agg(h2, src, dst, z2)
    return _dense(h2, acc3, degacc, W_self3, W_neigh3, b3, relu=False)


# speedup vs baseline: 10.3939x; 1.0084x over previous
import functools

import jax
import jax.numpy as jnp
from jax import lax
from jax.experimental import pallas as pl
from jax.experimental.pallas import tpu as pltpu
import jax.experimental.pallas.tpu_sc as plsc

N_NODES = 10000
N_EDGES = 320000
D = 128

NC = 2
NS = 16
NW = NC * NS
EPW = N_EDGES // NW
K = 50
G = 5
CPG = 40
NBUF = 4
CP = 624
TAIL = N_NODES - NS * CP


def _zero_acc(z2_hbm, acc_sh, s):
    pltpu.sync_copy(z2_hbm.at[pl.ds(0, CP)], acc_sh.at[pl.ds(s * CP, CP)])

    @pl.when(s == NS - 1)
    def _():
        pltpu.sync_copy(z2_hbm.at[pl.ds(CP, TAIL)],
                        acc_sh.at[pl.ds(NS * CP, TAIL)])


def _drain_acc(acc_sh, out_hbm, c, s):
    pltpu.sync_copy(acc_sh.at[pl.ds(s * CP, CP)],
                    out_hbm.at[c, pl.ds(s * CP, CP)])

    @pl.when(s == NS - 1)
    def _():
        pltpu.sync_copy(acc_sh.at[pl.ds(NS * CP, TAIL)],
                        out_hbm.at[c, pl.ds(NS * CP, TAIL)])


def _agg_core(h_hbm, src_hbm, dst_hbm, acc_sh, sidx, didx, bufs,
              isS, isD, wid):
    def gather(si, i, rows, sem):
        return pltpu.async_copy(h_hbm.at[si.at[i]], rows, sem)

    def wait_gather(si, i, rows, sem):
        pltpu.make_async_copy(h_hbm.at[si.at[i]], rows, sem).wait()

    def scatter(di, i, rows, sem):
        return pltpu.async_copy(rows, acc_sh.at[di.at[i]], sem, add=True)

    def wait_scatter(di, i, rows, sem):
        pltpu.make_async_copy(rows, acc_sh.at[di.at[i]], sem).wait()

    pltpu.async_copy(src_hbm.at[wid, 0], sidx[0], isS)
    pltpu.async_copy(dst_hbm.at[wid, 0], didx[0], isD)

    for g in range(G):
        si = sidx[g % 2]
        di = didx[g % 2]
        pltpu.make_async_copy(src_hbm.at[wid, g], si, isS).wait()
        pltpu.make_async_copy(dst_hbm.at[wid, g], di, isD).wait()
        if g + 1 < G:
            pltpu.async_copy(src_hbm.at[wid, g + 1], sidx[(g + 1) % 2], isS)
            pltpu.async_copy(dst_hbm.at[wid, g + 1], didx[(g + 1) % 2], isD)

        for b, (rows, gsem, _) in enumerate(bufs):
            gather(si, b, rows, gsem)

        def body(j, carry, si=si, di=di):
            c0 = NBUF * j
            for b, (rows, gsem, ssem) in enumerate(bufs):
                wait_gather(si, c0 + b, rows, gsem)
                scatter(di, c0 + b, rows, ssem)
            for b, (rows, gsem, ssem) in enumerate(bufs):
                wait_scatter(di, c0 + b, rows, ssem)

                @pl.when(c0 + b + NBUF < CPG)
                def _(b=b, rows=rows, gsem=gsem):
                    gather(si, c0 + b + NBUF, rows, gsem)
            return carry

        lax.fori_loop(0, CPG // NBUF, body, 0)


def _sc_agg_body(h_hbm, src_hbm, dst_hbm, z2_hbm, out_hbm, acc_sh,
                 sidxA, sidxB, didxA, didxB, rows0, rows1, rows2, rows3,
                 isS, isD, gs0, gs1, gs2, gs3, ss0, ss1, ss2, ss3):
    c = lax.axis_index("c")
    s = lax.axis_index("s")
    wid = s * NC + c

    _zero_acc(z2_hbm, acc_sh, s)
    plsc.subcore_barrier()
    bufs = ((rows0, gs0, ss0), (rows1, gs1, ss1),
            (rows2, gs2, ss2), (rows3, gs3, ss3))
    _agg_core(h_hbm, src_hbm, dst_hbm, acc_sh, (sidxA, sidxB),
              (didxA, didxB), bufs, isS, isD, wid)
    plsc.subcore_barrier()
    _drain_acc(acc_sh, out_hbm, c, s)


def _deg_core(dst_hbm, ones_hbm, acc_sh, didx_all, rows, ss0, ss1, wid):
    pltpu.sync_copy(ones_hbm, rows)

    def scatter(i, sem):
        return pltpu.async_copy(rows, acc_sh.at[didx_all.at[i]], sem, add=True)

    def wait_scatter(i, sem):
        pltpu.make_async_copy(rows, acc_sh.at[didx_all.at[i]], sem).wait()

    def group(g, carry):
        pltpu.sync_copy(dst_hbm.at[wid, g], didx_all)

        def body(j, carry2):
            a = 2 * j
            b = a + 1
            scatter(a, ss0)
            scatter(b, ss1)
            wait_scatter(a, ss0)
            wait_scatter(b, ss1)
            return carry2

        lax.fori_loop(0, CPG // 2, body, 0)
        return carry

    lax.fori_loop(0, G, group, 0)


def _sc_degagg_body(h_hbm, src_hbm, dst_hbm, z2_hbm, ones_hbm,
                    out_hbm, deg_hbm, acc_sh,
                    sidxA, sidxB, didxA, didxB, rows0, rows1, rows2, rows3,
                    isS, isD, gs0, gs1, gs2, gs3, ss0, ss1, ss2, ss3):
    c = lax.axis_index("c")
    s = lax.axis_index("s")
    wid = s * NC + c

    _zero_acc(z2_hbm, acc_sh, s)
    plsc.subcore_barrier()
    _deg_core(dst_hbm, ones_hbm, acc_sh, didxA, rows0, ss0, ss1, wid)
    plsc.subcore_barrier()
    _drain_acc(acc_sh, deg_hbm, c, s)
    _zero_acc(z2_hbm, acc_sh, s)
    plsc.subcore_barrier()
    bufs = ((rows0, gs0, ss0), (rows1, gs1, ss1),
            (rows2, gs2, ss2), (rows3, gs3, ss3))
    _agg_core(h_hbm, src_hbm, dst_hbm, acc_sh, (sidxA, sidxB),
              (didxA, didxB), bufs, isS, isD, wid)
    plsc.subcore_barrier()
    _drain_acc(acc_sh, out_hbm, c, s)


_MESH = dict(core_axis_name="c", subcore_axis_name="s")
_ACC_OUT = [jax.ShapeDtypeStruct((NC, N_NODES, D), jnp.float32)]


def _make_sc_agg():
    return pl.kernel(
        _sc_agg_body,
        out_type=_ACC_OUT,
        mesh=plsc.VectorSubcoreMesh(**_MESH),
        scratch_types=[
            pltpu.VMEM_SHARED((N_NODES, D), jnp.float32),
            pltpu.VMEM((CPG, K), jnp.int32),
            pltpu.VMEM((CPG, K), jnp.int32),
            pltpu.VMEM((CPG, K), jnp.int32),
            pltpu.VMEM((CPG, K), jnp.int32),
            pltpu.VMEM((K, D), jnp.float32),
            pltpu.VMEM((K, D), jnp.float32),
            pltpu.VMEM((K, D), jnp.float32),
            pltpu.VMEM((K, D), jnp.float32),
        ] + [pltpu.SemaphoreType.DMA] * 10)


def _make_sc_degagg():
    return pl.kernel(
        _sc_degagg_body,
        out_type=_ACC_OUT * 2,
        mesh=plsc.VectorSubcoreMesh(**_MESH),
        scratch_types=[
            pltpu.VMEM_SHARED((N_NODES, D), jnp.float32),
            pltpu.VMEM((CPG, K), jnp.int32),
            pltpu.VMEM((CPG, K), jnp.int32),
            pltpu.VMEM((CPG, K), jnp.int32),
            pltpu.VMEM((CPG, K), jnp.int32),
            pltpu.VMEM((K, D), jnp.float32),
            pltpu.VMEM((K, D), jnp.float32),
            pltpu.VMEM((K, D), jnp.float32),
            pltpu.VMEM((K, D), jnp.float32),
        ] + [pltpu.SemaphoreType.DMA] * 10)


def _dense_body(relu, h_ref, acc_ref, deg_ref, ws_ref, wn_ref, b_ref, o_ref):
    h = h_ref[...]
    a = acc_ref[0] + acc_ref[1]
    dsum = deg_ref[0, :, 0] + deg_ref[1, :, 0]
    r = (1.0 / jnp.maximum(dsum, 1.0))[:, None]
    hn = a * r
    out = (jnp.dot(h, ws_ref[...], preferred_element_type=jnp.float32)
           + jnp.dot(hn, wn_ref[...], preferred_element_type=jnp.float32)
           + b_ref[...][None, :])
    o_ref[...] = jnp.maximum(out, 0.0) if relu else out


def _dense(h, acc, deg, W_self, W_neigh, b, relu):
    R = 1000
    grid = (N_NODES // R,)
    return pl.pallas_call(
        functools.partial(_dense_body, relu),
        grid=grid,
        in_specs=[
            pl.BlockSpec((R, D), lambda i: (i, 0)),
            pl.BlockSpec((NC, R, D), lambda i: (0, i, 0)),
            pl.BlockSpec((NC, R, D), lambda i: (0, i, 0)),
            pl.BlockSpec((D, D), lambda i: (0, 0)),
            pl.BlockSpec((D, D), lambda i: (0, 0)),
            pl.BlockSpec((D,), lambda i: (0,)),
        ],
        out_specs=pl.BlockSpec((R, D), lambda i: (i, 0)),
        out_shape=jax.ShapeDtypeStruct((N_NODES, D), jnp.float32),
    )(h, acc, deg, W_self, W_neigh, b)


def kernel(x, edge_index, W_neigh1, W_self1, b1, W_neigh2, W_self2, b2,
           W_neigh3, W_self3, b3):
    src = edge_index[0].reshape(NW, G, CPG, K)
    dst = edge_index[1].reshape(NW, G, CPG, K)
    z2 = jnp.zeros((CP + TAIL, D), jnp.float32)
    ones_rows = jnp.ones((K, D), jnp.float32)

    agg = _make_sc_agg()
    degagg = _make_sc_degagg()

    acc1, degacc = degagg(x, src, dst, z2, ones_rows)
    h1 = _dense(x, acc1, degacc, W_self1, W_neigh1, b1, relu=True)
    acc2, = agg(h1, src, dst, z2)
    h2 = _dense(h1, acc2, degacc, W_self2, W_neigh2, b2, relu=True)
    acc3, = agg(h2, src, dst, z2)
    return _dense(h2, acc3, degacc, W_self3, W_neigh3, b3, relu=False)


# speedup vs baseline: 10.5209x; 1.0122x over previous
import functools

import jax
import jax.numpy as jnp
from jax import lax
from jax.experimental import pallas as pl
from jax.experimental.pallas import tpu as pltpu
import jax.experimental.pallas.tpu_sc as plsc

N_NODES = 10000
N_EDGES = 320000
D = 128

NC = 2
NS = 16
NW = NC * NS
EPW = N_EDGES // NW
K = 50
G = 5
CPG = 40
NBUF = 4
CP = 624
TAIL = N_NODES - NS * CP


def _zero_acc(z2_hbm, acc_sh, s):
    pltpu.sync_copy(z2_hbm.at[pl.ds(0, CP)], acc_sh.at[pl.ds(s * CP, CP)])

    @pl.when(s == NS - 1)
    def _():
        pltpu.sync_copy(z2_hbm.at[pl.ds(CP, TAIL)],
                        acc_sh.at[pl.ds(NS * CP, TAIL)])


def _drain_acc(acc_sh, out_hbm, c, s):
    pltpu.sync_copy(acc_sh.at[pl.ds(s * CP, CP)],
                    out_hbm.at[c, pl.ds(s * CP, CP)])

    @pl.when(s == NS - 1)
    def _():
        pltpu.sync_copy(acc_sh.at[pl.ds(NS * CP, TAIL)],
                        out_hbm.at[c, pl.ds(NS * CP, TAIL)])


def _agg_core(h_hbm, src_hbm, dst_hbm, acc_sh, sidx, didx, bufs,
              isS, isD, wid):
    def gather(si, i, rows, sem):
        return pltpu.async_copy(h_hbm.at[si.at[i]], rows, sem)

    def wait_gather(si, i, rows, sem):
        pltpu.make_async_copy(h_hbm.at[si.at[i]], rows, sem).wait()

    def scatter(di, i, rows, sem):
        return pltpu.async_copy(rows, acc_sh.at[di.at[i]], sem, add=True)

    def wait_scatter(di, i, rows, sem):
        pltpu.make_async_copy(rows, acc_sh.at[di.at[i]], sem).wait()

    pltpu.async_copy(src_hbm.at[wid, 0], sidx[0], isS)
    pltpu.async_copy(dst_hbm.at[wid, 0], didx[0], isD)
    pltpu.make_async_copy(src_hbm.at[wid, 0], sidx[0], isS).wait()
    pltpu.make_async_copy(dst_hbm.at[wid, 0], didx[0], isD).wait()
    if G > 1:
        pltpu.async_copy(src_hbm.at[wid, 1], sidx[1], isS)
        pltpu.async_copy(dst_hbm.at[wid, 1], didx[1], isD)
    for b, (rows, gsem, _) in enumerate(bufs):
        gather(sidx[0], b, rows, gsem)

    for g in range(G):
        si = sidx[g % 2]
        di = didx[g % 2]

        def body(j, carry, si=si, di=di):
            c0 = NBUF * j
            for b, (rows, gsem, ssem) in enumerate(bufs):
                wait_gather(si, c0 + b, rows, gsem)
                scatter(di, c0 + b, rows, ssem)
            for b, (rows, gsem, ssem) in enumerate(bufs):
                wait_scatter(di, c0 + b, rows, ssem)
                gather(si, c0 + b + NBUF, rows, gsem)
            return carry

        lax.fori_loop(0, CPG // NBUF - 1, body, 0)

        c0 = CPG - NBUF
        for b, (rows, gsem, ssem) in enumerate(bufs):
            wait_gather(si, c0 + b, rows, gsem)
            scatter(di, c0 + b, rows, ssem)
        if g + 1 < G:
            si_n = sidx[(g + 1) % 2]
            pltpu.make_async_copy(src_hbm.at[wid, g + 1], si_n, isS).wait()
            pltpu.make_async_copy(dst_hbm.at[wid, g + 1],
                                  didx[(g + 1) % 2], isD).wait()
            for b, (rows, gsem, ssem) in enumerate(bufs):
                wait_scatter(di, c0 + b, rows, ssem)
                gather(si_n, b, rows, gsem)
            if g + 2 < G:
                pltpu.async_copy(src_hbm.at[wid, g + 2], sidx[g % 2], isS)
                pltpu.async_copy(dst_hbm.at[wid, g + 2], didx[g % 2], isD)
        else:
            for b, (rows, gsem, ssem) in enumerate(bufs):
                wait_scatter(di, c0 + b, rows, ssem)


def _sc_agg_body(h_hbm, src_hbm, dst_hbm, z2_hbm, out_hbm, acc_sh,
                 sidxA, sidxB, didxA, didxB, rows0, rows1, rows2, rows3,
                 isS, isD, gs0, gs1, gs2, gs3, ss0, ss1, ss2, ss3):
    c = lax.axis_index("c")
    s = lax.axis_index("s")
    wid = s * NC + c

    _zero_acc(z2_hbm, acc_sh, s)
    plsc.subcore_barrier()
    bufs = ((rows0, gs0, ss0), (rows1, gs1, ss1),
            (rows2, gs2, ss2), (rows3, gs3, ss3))
    _agg_core(h_hbm, src_hbm, dst_hbm, acc_sh, (sidxA, sidxB),
              (didxA, didxB), bufs, isS, isD, wid)
    plsc.subcore_barrier()
    _drain_acc(acc_sh, out_hbm, c, s)


def _deg_core(dst_hbm, ones_hbm, acc_sh, didx_all, rows, ss0, ss1, wid):
    pltpu.sync_copy(ones_hbm, rows)

    def scatter(i, sem):
        return pltpu.async_copy(rows, acc_sh.at[didx_all.at[i]], sem, add=True)

    def wait_scatter(i, sem):
        pltpu.make_async_copy(rows, acc_sh.at[didx_all.at[i]], sem).wait()

    def group(g, carry):
        pltpu.sync_copy(dst_hbm.at[wid, g], didx_all)

        def body(j, carry2):
            a = 2 * j
            b = a + 1
            scatter(a, ss0)
            scatter(b, ss1)
            wait_scatter(a, ss0)
            wait_scatter(b, ss1)
            return carry2

        lax.fori_loop(0, CPG // 2, body, 0)
        return carry

    lax.fori_loop(0, G, group, 0)


def _sc_degagg_body(h_hbm, src_hbm, dst_hbm, z2_hbm, ones_hbm,
                    out_hbm, deg_hbm, acc_sh,
                    sidxA, sidxB, didxA, didxB, rows0, rows1, rows2, rows3,
                    isS, isD, gs0, gs1, gs2, gs3, ss0, ss1, ss2, ss3):
    c = lax.axis_index("c")
    s = lax.axis_index("s")
    wid = s * NC + c

    _zero_acc(z2_hbm, acc_sh, s)
    plsc.subcore_barrier()
    _deg_core(dst_hbm, ones_hbm, acc_sh, didxA, rows0, ss0, ss1, wid)
    plsc.subcore_barrier()
    _drain_acc(acc_sh, deg_hbm, c, s)
    _zero_acc(z2_hbm, acc_sh, s)
    plsc.subcore_barrier()
    bufs = ((rows0, gs0, ss0), (rows1, gs1, ss1),
            (rows2, gs2, ss2), (rows3, gs3, ss3))
    _agg_core(h_hbm, src_hbm, dst_hbm, acc_sh, (sidxA, sidxB),
              (didxA, didxB), bufs, isS, isD, wid)
    plsc.subcore_barrier()
    _drain_acc(acc_sh, out_hbm, c, s)


_MESH = dict(core_axis_name="c", subcore_axis_name="s")
_ACC_OUT = [jax.ShapeDtypeStruct((NC, N_NODES, D), jnp.float32)]


def _make_sc_agg():
    return pl.kernel(
        _sc_agg_body,
        out_type=_ACC_OUT,
        mesh=plsc.VectorSubcoreMesh(**_MESH),
        scratch_types=[
            pltpu.VMEM_SHARED((N_NODES, D), jnp.float32),
            pltpu.VMEM((CPG, K), jnp.int32),
            pltpu.VMEM((CPG, K), jnp.int32),
            pltpu.VMEM((CPG, K), jnp.int32),
            pltpu.VMEM((CPG, K), jnp.int32),
            pltpu.VMEM((K, D), jnp.float32),
            pltpu.VMEM((K, D), jnp.float32),
            pltpu.VMEM((K, D), jnp.float32),
            pltpu.VMEM((K, D), jnp.float32),
        ] + [pltpu.SemaphoreType.DMA] * 10)


def _make_sc_degagg():
    return pl.kernel(
        _sc_degagg_body,
        out_type=_ACC_OUT * 2,
        mesh=plsc.VectorSubcoreMesh(**_MESH),
        scratch_types=[
            pltpu.VMEM_SHARED((N_NODES, D), jnp.float32),
            pltpu.VMEM((CPG, K), jnp.int32),
            pltpu.VMEM((CPG, K), jnp.int32),
            pltpu.VMEM((CPG, K), jnp.int32),
            pltpu.VMEM((CPG, K), jnp.int32),
            pltpu.VMEM((K, D), jnp.float32),
            pltpu.VMEM((K, D), jnp.float32),
            pltpu.VMEM((K, D), jnp.float32),
            pltpu.VMEM((K, D), jnp.float32),
        ] + [pltpu.SemaphoreType.DMA] * 10)


def _dense_body(relu, h_ref, acc_ref, deg_ref, ws_ref, wn_ref, b_ref, o_ref):
    h = h_ref[...]
    a = acc_ref[0] + acc_ref[1]
    dsum = deg_ref[0, :, 0] + deg_ref[1, :, 0]
    r = (1.0 / jnp.maximum(dsum, 1.0))[:, None]
    hn = a * r
    out = (jnp.dot(h, ws_ref[...], preferred_element_type=jnp.float32)
           + jnp.dot(hn, wn_ref[...], preferred_element_type=jnp.float32)
           + b_ref[...][None, :])
    o_ref[...] = jnp.maximum(out, 0.0) if relu else out


def _dense(h, acc, deg, W_self, W_neigh, b, relu):
    R = 1000
    grid = (N_NODES // R,)
    return pl.pallas_call(
        functools.partial(_dense_body, relu),
        grid=grid,
        in_specs=[
            pl.BlockSpec((R, D), lambda i: (i, 0)),
            pl.BlockSpec((NC, R, D), lambda i: (0, i, 0)),
            pl.BlockSpec((NC, R, D), lambda i: (0, i, 0)),
            pl.BlockSpec((D, D), lambda i: (0, 0)),
            pl.BlockSpec((D, D), lambda i: (0, 0)),
            pl.BlockSpec((D,), lambda i: (0,)),
        ],
        out_specs=pl.BlockSpec((R, D), lambda i: (i, 0)),
        out_shape=jax.ShapeDtypeStruct((N_NODES, D), jnp.float32),
    )(h, acc, deg, W_self, W_neigh, b)


def kernel(x, edge_index, W_neigh1, W_self1, b1, W_neigh2, W_self2, b2,
           W_neigh3, W_self3, b3):
    src = edge_index[0].reshape(NW, G, CPG, K)
    dst = edge_index[1].reshape(NW, G, CPG, K)
    z2 = jnp.zeros((CP + TAIL, D), jnp.float32)
    ones_rows = jnp.ones((K, D), jnp.float32)

    agg = _make_sc_agg()
    degagg = _make_sc_degagg()

    acc1, degacc = degagg(x, src, dst, z2, ones_rows)
    h1 = _dense(x, acc1, degacc, W_self1, W_neigh1, b1, relu=True)
    acc2, = agg(h1, src, dst, z2)
    h2 = _dense(h1, acc2, degacc, W_self2, W_neigh2, b2, relu=True)
    acc3, = agg(h2, src, dst, z2)
    return _dense(h2, acc3, degacc, W_self3, W_neigh3, b3, relu=False)


# speedup vs baseline: 10.5824x; 1.0058x over previous
import functools

import jax
import jax.numpy as jnp
from jax import lax
from jax.experimental import pallas as pl
from jax.experimental.pallas import tpu as pltpu
import jax.experimental.pallas.tpu_sc as plsc

N_NODES = 10000
N_EDGES = 320000
D = 128

NC = 2
NS = 16
NW = NC * NS
EPW = N_EDGES // NW
K = 50
G = 5
CPG = 40
NBUF = 4
CP = 624
TAIL = N_NODES - NS * CP


def _zero_acc(z2_hbm, acc_sh, s):
    pltpu.sync_copy(z2_hbm.at[pl.ds(0, CP)], acc_sh.at[pl.ds(s * CP, CP)])

    @pl.when(s == NS - 1)
    def _():
        pltpu.sync_copy(z2_hbm.at[pl.ds(CP, TAIL)],
                        acc_sh.at[pl.ds(NS * CP, TAIL)])


def _drain_acc(acc_sh, out_hbm, c, s):
    pltpu.sync_copy(acc_sh.at[pl.ds(s * CP, CP)],
                    out_hbm.at[c, pl.ds(s * CP, CP)])

    @pl.when(s == NS - 1)
    def _():
        pltpu.sync_copy(acc_sh.at[pl.ds(NS * CP, TAIL)],
                        out_hbm.at[c, pl.ds(NS * CP, TAIL)])


def _agg_core(h_hbm, src_hbm, dst_hbm, acc_sh, sidx, didx, bufs,
              isS, isD, wid):
    def gather(si, i, rows, sem):
        return pltpu.async_copy(h_hbm.at[si.at[i]], rows, sem)

    def wait_gather(si, i, rows, sem):
        pltpu.make_async_copy(h_hbm.at[si.at[i]], rows, sem).wait()

    def scatter(di, i, rows, sem):
        return pltpu.async_copy(rows, acc_sh.at[di.at[i]], sem, add=True)

    def wait_scatter(di, i, rows, sem):
        pltpu.make_async_copy(rows, acc_sh.at[di.at[i]], sem).wait()

    pltpu.async_copy(src_hbm.at[wid, 0], sidx[0], isS)
    pltpu.async_copy(dst_hbm.at[wid, 0], didx[0], isD)
    pltpu.make_async_copy(src_hbm.at[wid, 0], sidx[0], isS).wait()
    pltpu.make_async_copy(dst_hbm.at[wid, 0], didx[0], isD).wait()
    if G > 1:
        pltpu.async_copy(src_hbm.at[wid, 1], sidx[1], isS)
        pltpu.async_copy(dst_hbm.at[wid, 1], didx[1], isD)
    for b, (rows, gsem, _) in enumerate(bufs):
        gather(sidx[0], b, rows, gsem)

    for g in range(G):
        si = sidx[g % 2]
        di = didx[g % 2]

        def body(j, carry, si=si, di=di):
            c0 = NBUF * j
            for b, (rows, gsem, ssem) in enumerate(bufs):
                wait_gather(si, c0 + b, rows, gsem)
                scatter(di, c0 + b, rows, ssem)
            for b, (rows, gsem, ssem) in enumerate(bufs):
                wait_scatter(di, c0 + b, rows, ssem)
                gather(si, c0 + b + NBUF, rows, gsem)
            return carry

        lax.fori_loop(0, CPG // NBUF - 1, body, 0)

        c0 = CPG - NBUF
        for b, (rows, gsem, ssem) in enumerate(bufs):
            wait_gather(si, c0 + b, rows, gsem)
            scatter(di, c0 + b, rows, ssem)
        if g + 1 < G:
            si_n = sidx[(g + 1) % 2]
            pltpu.make_async_copy(src_hbm.at[wid, g + 1], si_n, isS).wait()
            pltpu.make_async_copy(dst_hbm.at[wid, g + 1],
                                  didx[(g + 1) % 2], isD).wait()
            for b, (rows, gsem, ssem) in enumerate(bufs):
                wait_scatter(di, c0 + b, rows, ssem)
                gather(si_n, b, rows, gsem)
            if g + 2 < G:
                pltpu.async_copy(src_hbm.at[wid, g + 2], sidx[g % 2], isS)
                pltpu.async_copy(dst_hbm.at[wid, g + 2], didx[g % 2], isD)
        else:
            for b, (rows, gsem, ssem) in enumerate(bufs):
                wait_scatter(di, c0 + b, rows, ssem)


def _sc_agg_body(h_hbm, src_hbm, dst_hbm, z2_hbm, out_hbm, acc_sh,
                 sidxA, sidxB, didxA, didxB, rows0, rows1, rows2, rows3,
                 isS, isD, gs0, gs1, gs2, gs3, ss0, ss1, ss2, ss3):
    c = lax.axis_index("c")
    s = lax.axis_index("s")
    wid = s * NC + c

    _zero_acc(z2_hbm, acc_sh, s)
    plsc.subcore_barrier()
    bufs = ((rows0, gs0, ss0), (rows1, gs1, ss1),
            (rows2, gs2, ss2), (rows3, gs3, ss3))
    _agg_core(h_hbm, src_hbm, dst_hbm, acc_sh, (sidxA, sidxB),
              (didxA, didxB), bufs, isS, isD, wid)
    plsc.subcore_barrier()
    _drain_acc(acc_sh, out_hbm, c, s)


def _deg_core(dst_hbm, ones_hbm, acc_sh, didx, rows, ssems, isD, wid):
    pltpu.sync_copy(ones_hbm, rows)

    def scatter(di, i, sem):
        return pltpu.async_copy(rows, acc_sh.at[di.at[i]], sem, add=True)

    def wait_scatter(di, i, sem):
        pltpu.make_async_copy(rows, acc_sh.at[di.at[i]], sem).wait()

    pltpu.async_copy(dst_hbm.at[wid, 0], didx[0], isD)
    for g in range(G):
        di = didx[g % 2]
        pltpu.make_async_copy(dst_hbm.at[wid, g], di, isD).wait()
        if g + 1 < G:
            pltpu.async_copy(dst_hbm.at[wid, g + 1], didx[(g + 1) % 2], isD)

        def body(j, carry, di=di):
            c0 = NBUF * j
            for b, sem in enumerate(ssems):
                scatter(di, c0 + b, sem)
            for b, sem in enumerate(ssems):
                wait_scatter(di, c0 + b, sem)
            return carry

        lax.fori_loop(0, CPG // NBUF, body, 0)


def _sc_degagg_body(h_hbm, src_hbm, dst_hbm, z2_hbm, ones_hbm,
                    out_hbm, deg_hbm, acc_sh,
                    sidxA, sidxB, didxA, didxB, rows0, rows1, rows2, rows3,
                    isS, isD, gs0, gs1, gs2, gs3, ss0, ss1, ss2, ss3):
    c = lax.axis_index("c")
    s = lax.axis_index("s")
    wid = s * NC + c

    _zero_acc(z2_hbm, acc_sh, s)
    plsc.subcore_barrier()
    _deg_core(dst_hbm, ones_hbm, acc_sh, (didxA, didxB), rows0,
              (ss0, ss1, ss2, ss3), isD, wid)
    plsc.subcore_barrier()
    _drain_acc(acc_sh, deg_hbm, c, s)
    _zero_acc(z2_hbm, acc_sh, s)
    plsc.subcore_barrier()
    bufs = ((rows0, gs0, ss0), (rows1, gs1, ss1),
            (rows2, gs2, ss2), (rows3, gs3, ss3))
    _agg_core(h_hbm, src_hbm, dst_hbm, acc_sh, (sidxA, sidxB),
              (didxA, didxB), bufs, isS, isD, wid)
    plsc.subcore_barrier()
    _drain_acc(acc_sh, out_hbm, c, s)


_MESH = dict(core_axis_name="c", subcore_axis_name="s")
_ACC_OUT = [jax.ShapeDtypeStruct((NC, N_NODES, D), jnp.float32)]


def _make_sc_agg():
    return pl.kernel(
        _sc_agg_body,
        out_type=_ACC_OUT,
        mesh=plsc.VectorSubcoreMesh(**_MESH),
        scratch_types=[
            pltpu.VMEM_SHARED((N_NODES, D), jnp.float32),
            pltpu.VMEM((CPG, K), jnp.int32),
            pltpu.VMEM((CPG, K), jnp.int32),
            pltpu.VMEM((CPG, K), jnp.int32),
            pltpu.VMEM((CPG, K), jnp.int32),
            pltpu.VMEM((K, D), jnp.float32),
            pltpu.VMEM((K, D), jnp.float32),
            pltpu.VMEM((K, D), jnp.float32),
            pltpu.VMEM((K, D), jnp.float32),
        ] + [pltpu.SemaphoreType.DMA] * 10)


def _make_sc_degagg():
    return pl.kernel(
        _sc_degagg_body,
        out_type=_ACC_OUT * 2,
        mesh=plsc.VectorSubcoreMesh(**_MESH),
        scratch_types=[
            pltpu.VMEM_SHARED((N_NODES, D), jnp.float32),
            pltpu.VMEM((CPG, K), jnp.int32),
            pltpu.VMEM((CPG, K), jnp.int32),
            pltpu.VMEM((CPG, K), jnp.int32),
            pltpu.VMEM((CPG, K), jnp.int32),
            pltpu.VMEM((K, D), jnp.float32),
            pltpu.VMEM((K, D), jnp.float32),
            pltpu.VMEM((K, D), jnp.float32),
            pltpu.VMEM((K, D), jnp.float32),
        ] + [pltpu.SemaphoreType.DMA] * 10)


def _dense_body(relu, h_ref, acc_ref, deg_ref, ws_ref, wn_ref, b_ref, o_ref):
    h = h_ref[...]
    a = acc_ref[0] + acc_ref[1]
    dsum = deg_ref[0, :, 0] + deg_ref[1, :, 0]
    r = (1.0 / jnp.maximum(dsum, 1.0))[:, None]
    hn = a * r
    out = (jnp.dot(h, ws_ref[...], preferred_element_type=jnp.float32)
           + jnp.dot(hn, wn_ref[...], preferred_element_type=jnp.float32)
           + b_ref[...][None, :])
    o_ref[...] = jnp.maximum(out, 0.0) if relu else out


def _dense(h, acc, deg, W_self, W_neigh, b, relu):
    R = 1000
    grid = (N_NODES // R,)
    return pl.pallas_call(
        functools.partial(_dense_body, relu),
        grid=grid,
        in_specs=[
            pl.BlockSpec((R, D), lambda i: (i, 0)),
            pl.BlockSpec((NC, R, D), lambda i: (0, i, 0)),
            pl.BlockSpec((NC, R, D), lambda i: (0, i, 0)),
            pl.BlockSpec((D, D), lambda i: (0, 0)),
            pl.BlockSpec((D, D), lambda i: (0, 0)),
            pl.BlockSpec((D,), lambda i: (0,)),
        ],
        out_specs=pl.BlockSpec((R, D), lambda i: (i, 0)),
        out_shape=jax.ShapeDtypeStruct((N_NODES, D), jnp.float32),
    )(h, acc, deg, W_self, W_neigh, b)


def kernel(x, edge_index, W_neigh1, W_self1, b1, W_neigh2, W_self2, b2,
           W_neigh3, W_self3, b3):
    src = edge_index[0].reshape(NW, G, CPG, K)
    dst = edge_index[1].reshape(NW, G, CPG, K)
    z2 = jnp.zeros((CP + TAIL, D), jnp.float32)
    ones_rows = jnp.ones((K, D), jnp.float32)

    agg = _make_sc_agg()
    degagg = _make_sc_degagg()

    acc1, degacc = degagg(x, src, dst, z2, ones_rows)
    h1 = _dense(x, acc1, degacc, W_self1, W_neigh1, b1, relu=True)
    acc2, = agg(h1, src, dst, z2)
    h2 = _dense(h1, acc2, degacc, W_self2, W_neigh2, b2, relu=True)
    acc3, = agg(h2, src, dst, z2)
    return _dense(h2, acc3, degacc, W_self3, W_neigh3, b3, relu=False)
